# baseline (speedup 1.0000x reference)
import functools

import jax
import jax.numpy as jnp
from jax import lax
from jax.experimental import pallas as pl
from jax.experimental.pallas import tpu as pltpu
from jax.experimental.pallas import tpu_sc as plsc

_NC = 2
_NS = 16
_NW = _NC * _NS

_IDX_CHUNK = 128
_HALF = 512
_SUPER = 1024


def _sc_gather(table, idx):
    M = idx.shape[0]
    C = table.shape[1]
    per_w = M // _NW
    supers = per_w // _SUPER
    mesh = plsc.VectorSubcoreMesh(
        core_axis_name="c", subcore_axis_name="s",
        num_cores=_NC, num_subcores=_NS)

    @functools.partial(
        pl.kernel,
        out_type=jax.ShapeDtypeStruct((M, C), jnp.float32),
        mesh=mesh,
        scratch_types=[
            pltpu.VMEM((_SUPER,), jnp.int32),
            pltpu.VMEM((_HALF, C), jnp.float32),
            pltpu.SemaphoreType.DMA,
        ],
        compiler_params=pltpu.CompilerParams(use_tc_tiling_on_sc=False),
    )
    def gather_kernel(table_hbm, idx_hbm, out_hbm, idx_v, rows_v, sem):
        wid = lax.axis_index("s") * _NC + lax.axis_index("c")
        base = wid * per_w

        def body(i, carry):
            i0 = pl.multiple_of(base + i * _SUPER, _SUPER)
            pltpu.sync_copy(idx_hbm.at[pl.ds(i0, _SUPER)], idx_v)
            for half in range(_SUPER // _HALF):
                row0 = pl.multiple_of(base + i * _SUPER + half * _HALF, _HALF)
                copies = []
                for j in range(_HALF // _IDX_CHUNK):
                    copies.append(pltpu.async_copy(
                        table_hbm.at[idx_v.at[pl.ds(
                            (half * (_HALF // _IDX_CHUNK) + j) * _IDX_CHUNK,
                            _IDX_CHUNK)]],
                        rows_v.at[pl.ds(j * _IDX_CHUNK, _IDX_CHUNK)],
                        sem))
                for cp in copies:
                    cp.wait()
                pltpu.sync_copy(rows_v, out_hbm.at[pl.ds(row0, _HALF)])
            return carry

        lax.fori_loop(0, supers, body, 0)

    return gather_kernel(table, idx)


def _tc_conv(xp, g, Ws, bias, skip, act):
    E, C = xp.shape
    F = Ws[0].shape[1]
    R = 2048
    grid = (E // R,)
    nb = E // R
    row_spec = pl.BlockSpec((R, C), lambda i: (i, 0))
    g_specs = [pl.BlockSpec((R, C), lambda i, k=k: (i + k * nb, 0))
               for k in range(4)]
    w_specs = [pl.BlockSpec((C, F), lambda i: (0, 0)) for _ in range(5)]
    b_spec = pl.BlockSpec((1, F), lambda i: (0, 0))
    out_spec = pl.BlockSpec((R, F), lambda i: (i, 0))

    def body(x_ref, a_ref, b_ref, c_ref, d_ref, w0, w1, w2, w3, w4,
             bias_ref, *rest):
        out_ref = rest[-1]
        xx = x_ref[...]
        a = a_ref[...]
        b = b_ref[...]
        c = c_ref[...]
        d = d_ref[...]
        dot = functools.partial(
            jnp.dot, preferred_element_type=jnp.float32)
        h = dot(xx, w0[...])
        h += dot(jnp.abs(a - c), w1[...])
        h += dot(a + c, w2[...])
        h += dot(jnp.abs(b - d), w3[...])
        h += dot(b + d, w4[...])
        h += bias_ref[...]
        if act:
            h = jnp.where(h >= 0, h, 0.1 * h)
        if len(rest) == 2:
            h += rest[0][...]
        out_ref[...] = h

    in_specs = [row_spec] + g_specs + w_specs + [b_spec]
    args = [xp, g, g, g, g] + list(Ws) + [bias.reshape(1, F)]
    if skip is not None:
        in_specs.append(pl.BlockSpec((R, F), lambda i: (i, 0)))
        args.append(skip)
    return pl.pallas_call(
        body,
        grid=grid,
        in_specs=in_specs,
        out_specs=out_spec,
        out_shape=jax.ShapeDtypeStruct((E, F), jnp.float32),
    )(*args)


def _layer(hp, idx, W, bias, skip, act):
    C = W.shape[0] // 5
    Ws = [W[k * C:(k + 1) * C] for k in range(5)]
    g = _sc_gather(hp, idx)
    return _tc_conv(hp, g, Ws, bias, skip, act)


def kernel(fixed_input_features, gemm_edges, We0, be0, We1, be1, We2, be2,
           Wd0, bd0, Wd1, bd1, Wd2, bd2, Wf, bf):
    E = fixed_input_features.shape[0]
    idx = gemm_edges.T.reshape(4 * E)

    s0 = fixed_input_features
    s1 = _layer(s0, idx, We0, be0, None, True)
    s2 = _layer(s1, idx, We1, be1, None, True)
    h = _layer(s2, idx, We2, be2, None, True)
    h = _layer(h, idx, Wd0, bd0, s2, True)
    h = _layer(h, idx, Wd1, bd1, s1, True)
    h = _layer(h, idx, Wd2, bd2, s0, True)
    return _layer(h, idx, Wf, bf, None, False)


# speedup vs baseline: 1.0135x; 1.0135x over previous
import functools

import jax
import jax.numpy as jnp
from jax import lax
from jax.experimental import pallas as pl
from jax.experimental.pallas import tpu as pltpu
from jax.experimental.pallas import tpu_sc as plsc

_NC = 2
_NS = 16
_NW = _NC * _NS

_IDX_CHUNK = 128
_HALF = 512
_SUPER = 1024


def _sc_gather(table, idx):
    M = idx.shape[0]
    C = table.shape[1]
    per_w = M // _NW
    supers = per_w // _SUPER
    mesh = plsc.VectorSubcoreMesh(
        core_axis_name="c", subcore_axis_name="s",
        num_cores=_NC, num_subcores=_NS)

    @functools.partial(
        pl.kernel,
        out_type=jax.ShapeDtypeStruct((M, C), jnp.float32),
        mesh=mesh,
        scratch_types=[
            pltpu.VMEM((_SUPER,), jnp.int32),
            pltpu.VMEM((_HALF, C), jnp.float32),
            pltpu.SemaphoreType.DMA,
        ],
        compiler_params=pltpu.CompilerParams(use_tc_tiling_on_sc=False),
    )
    def gather_kernel(table_hbm, idx_hbm, out_hbm, idx_v, rows_v, sem):
        wid = lax.axis_index("s") * _NC + lax.axis_index("c")
        base = wid * per_w

        def body(i, carry):
            i0 = pl.multiple_of(base + i * _SUPER, _SUPER)
            pltpu.sync_copy(idx_hbm.at[pl.ds(i0, _SUPER)], idx_v)
            for half in range(_SUPER // _HALF):
                row0 = pl.multiple_of(base + i * _SUPER + half * _HALF, _HALF)
                copies = []
                for j in range(_HALF // _IDX_CHUNK):
                    copies.append(pltpu.async_copy(
                        table_hbm.at[idx_v.at[pl.ds(
                            (half * (_HALF // _IDX_CHUNK) + j) * _IDX_CHUNK,
                            _IDX_CHUNK)]],
                        rows_v.at[pl.ds(j * _IDX_CHUNK, _IDX_CHUNK)],
                        sem))
                for cp in copies:
                    cp.wait()
                pltpu.sync_copy(rows_v, out_hbm.at[pl.ds(row0, _HALF)])
            return carry

        lax.fori_loop(0, supers, body, 0)

    return gather_kernel(table, idx)


def _tc_conv(xp, g, Ws, bias, skip, act):
    E, C = xp.shape
    F = Ws[0].shape[1]
    R = 2048
    grid = (E // R,)
    nb = E // R
    row_spec = pl.BlockSpec((R, C), lambda i: (i, 0))
    g_specs = [pl.BlockSpec((R, C), lambda i, k=k: (i + k * nb, 0))
               for k in range(4)]
    w_specs = [pl.BlockSpec((C, F), lambda i: (0, 0)) for _ in range(5)]
    b_spec = pl.BlockSpec((1, F), lambda i: (0, 0))
    out_spec = pl.BlockSpec((R, F), lambda i: (i, 0))

    def body(x_ref, a_ref, b_ref, c_ref, d_ref, w0, w1, w2, w3, w4,
             bias_ref, *rest):
        out_ref = rest[-1]
        xx = x_ref[...]
        a = a_ref[...]
        b = b_ref[...]
        c = c_ref[...]
        d = d_ref[...]
        dot = functools.partial(
            jnp.dot, preferred_element_type=jnp.float32)
        h = dot(xx, w0[...])
        h += dot(jnp.abs(a - c), w1[...])
        h += dot(a + c, w2[...])
        h += dot(jnp.abs(b - d), w3[...])
        h += dot(b + d, w4[...])
        h += bias_ref[...]
        if act:
            h = jnp.where(h >= 0, h, 0.1 * h)
        if len(rest) == 2:
            h += rest[0][...]
        out_ref[...] = h

    in_specs = [row_spec] + g_specs + w_specs + [b_spec]
    args = [xp, g, g, g, g] + list(Ws) + [bias.reshape(1, F)]
    if skip is not None:
        in_specs.append(pl.BlockSpec((R, F), lambda i: (i, 0)))
        args.append(skip)
    return pl.pallas_call(
        body,
        grid=grid,
        in_specs=in_specs,
        out_specs=out_spec,
        out_shape=jax.ShapeDtypeStruct((E, F), jnp.float32),
    )(*args)


def _layer(hp, idx, W, bias, skip, act, Fp=None):
    C = W.shape[0] // 5
    F = W.shape[1]
    Cp = hp.shape[1]
    if Fp is None:
        Fp = F
    Ws = [jnp.pad(W[k * C:(k + 1) * C], ((0, Cp - C), (0, Fp - F)))
          for k in range(5)]
    biasp = jnp.pad(bias, (0, Fp - F))
    g = _sc_gather(hp, idx)
    return _tc_conv(hp, g, Ws, biasp, skip, act)


def kernel(fixed_input_features, gemm_edges, We0, be0, We1, be1, We2, be2,
           Wd0, bd0, Wd1, bd1, Wd2, bd2, Wf, bf):
    E = fixed_input_features.shape[0]
    idx = gemm_edges.T.reshape(4 * E)

    s0 = jnp.pad(fixed_input_features, ((0, 0), (0, 2)))
    s1 = _layer(s0, idx, We0, be0, None, True)
    s2 = _layer(s1, idx, We1, be1, None, True)
    h = _layer(s2, idx, We2, be2, None, True)
    h = _layer(h, idx, Wd0, bd0, s2, True)
    h = _layer(h, idx, Wd1, bd1, s1, True)
    h = _layer(h, idx, Wd2, bd2, s0, True, Fp=8)
    return _layer(h, idx, Wf, bf, None, False)


# speedup vs baseline: 1.8974x; 1.8721x over previous
import functools

import jax
import jax.numpy as jnp
from jax import lax
from jax.experimental import pallas as pl
from jax.experimental.pallas import tpu as pltpu
from jax.experimental.pallas import tpu_sc as plsc

_NC = 2
_NS = 16
_NW = _NC * _NS

_LANES = 128
_IDX_CHUNK = 128
_HALF = 512
_SUPER = 1024


def _sc_gather(table_wide, idxq, Cp):
    M = idxq.shape[0]
    E = table_wide.shape[0]
    q = _LANES // Cp
    table = jnp.reshape(table_wide, (E * q, Cp))
    per_w = M // _NW
    supers = per_w // _SUPER
    mesh = plsc.VectorSubcoreMesh(
        core_axis_name="c", subcore_axis_name="s",
        num_cores=_NC, num_subcores=_NS)

    @functools.partial(
        pl.kernel,
        out_type=jax.ShapeDtypeStruct((M, _LANES), jnp.float32),
        mesh=mesh,
        scratch_types=[
            pltpu.VMEM((_SUPER,), jnp.int32),
            pltpu.VMEM((_HALF, Cp), jnp.float32),
            pltpu.SemaphoreType.DMA,
        ],
        compiler_params=pltpu.CompilerParams(use_tc_tiling_on_sc=False),
    )
    def gather_kernel(table_hbm, idx_hbm, out_hbm, idx_v, rows_v, sem):
        wid = lax.axis_index("s") * _NC + lax.axis_index("c")
        base = wid * per_w

        def body(i, carry):
            i0 = pl.multiple_of(base + i * _SUPER, _SUPER)
            pltpu.sync_copy(idx_hbm.at[pl.ds(i0, _SUPER)], idx_v)
            for half in range(_SUPER // _HALF):
                row0 = pl.multiple_of(base + i * _SUPER + half * _HALF, _HALF)
                copies = []
                for j in range(_HALF // _IDX_CHUNK):
                    copies.append(pltpu.async_copy(
                        table_hbm.at[idx_v.at[pl.ds(
                            (half * (_HALF // _IDX_CHUNK) + j) * _IDX_CHUNK,
                            _IDX_CHUNK)]],
                        rows_v.at[pl.ds(j * _IDX_CHUNK, _IDX_CHUNK)],
                        sem))
                for cp in copies:
                    cp.wait()
                pltpu.sync_copy(
                    rows_v,
                    out_hbm.at[pl.ds(row0, _HALF), pl.ds(0, Cp)])
            return carry

        lax.fori_loop(0, supers, body, 0)

    return gather_kernel(table, idxq)


def _tc_conv(xp, g, Ws, bias, skip, act, out_wide):
    E = xp.shape[0]
    Cp, Fp = Ws[0].shape
    R = 2048
    grid = (E // R,)
    nb = E // R
    row_spec = pl.BlockSpec((R, _LANES), lambda i: (i, 0))
    g_specs = [pl.BlockSpec((R, _LANES), lambda i, k=k: (i + k * nb, 0))
               for k in range(4)]
    w_specs = [pl.BlockSpec((Cp, Fp), lambda i: (0, 0)) for _ in range(5)]
    b_spec = pl.BlockSpec((1, Fp), lambda i: (0, 0))
    out_w = _LANES if out_wide else Fp
    out_spec = pl.BlockSpec((R, out_w), lambda i: (i, 0))
    out_shape = jax.ShapeDtypeStruct((E, out_w), jnp.float32)

    def body(x_ref, a_ref, b_ref, c_ref, d_ref, w0, w1, w2, w3, w4,
             bias_ref, *rest):
        out_ref = rest[-1]
        xx = x_ref[...][:, :Cp]
        a = a_ref[...][:, :Cp]
        b = b_ref[...][:, :Cp]
        c = c_ref[...][:, :Cp]
        d = d_ref[...][:, :Cp]
        dot = functools.partial(
            jnp.dot, preferred_element_type=jnp.float32)
        h = dot(xx, w0[...])
        h += dot(jnp.abs(a - c), w1[...])
        h += dot(a + c, w2[...])
        h += dot(jnp.abs(b - d), w3[...])
        h += dot(b + d, w4[...])
        h += bias_ref[...]
        if act:
            h = jnp.where(h >= 0, h, 0.1 * h)
        if len(rest) == 2:
            h += rest[0][...][:, :Fp]
        if out_wide:
            h = jnp.pad(h, ((0, 0), (0, _LANES - Fp)))
        out_ref[...] = h

    in_specs = [row_spec] + g_specs + w_specs + [b_spec]
    args = [xp, g, g, g, g] + list(Ws) + [bias.reshape(1, Fp)]
    if skip is not None:
        in_specs.append(pl.BlockSpec((R, _LANES), lambda i: (i, 0)))
        args.append(skip)
    return pl.pallas_call(
        body,
        grid=grid,
        in_specs=in_specs,
        out_specs=out_spec,
        out_shape=out_shape,
    )(*args)


def _layer(hp, idxs, W, bias, skip, act, Cp, Fp=None, out_wide=True):
    C = W.shape[0] // 5
    F = W.shape[1]
    if Fp is None:
        Fp = F
    Ws = [jnp.pad(W[k * C:(k + 1) * C], ((0, Cp - C), (0, Fp - F)))
          for k in range(5)]
    biasp = jnp.pad(bias, (0, Fp - F))
    g = _sc_gather(hp, idxs[Cp], Cp)
    return _tc_conv(hp, g, Ws, biasp, skip, act, out_wide)


def kernel(fixed_input_features, gemm_edges, We0, be0, We1, be1, We2, be2,
           Wd0, bd0, Wd1, bd1, Wd2, bd2, Wf, bf):
    E = fixed_input_features.shape[0]
    idx = gemm_edges.T.reshape(4 * E)
    idxs = {cp: idx * (_LANES // cp) for cp in (8, 16, 32, 64)}

    x0p = jnp.pad(fixed_input_features, ((0, 0), (0, _LANES - 6)))
    s1 = _layer(x0p, idxs, We0, be0, None, True, Cp=8)
    s2 = _layer(s1, idxs, We1, be1, None, True, Cp=16)
    h = _layer(s2, idxs, We2, be2, None, True, Cp=32)
    h = _layer(h, idxs, Wd0, bd0, s2, True, Cp=64)
    h = _layer(h, idxs, Wd1, bd1, s1, True, Cp=32)
    h = _layer(h, idxs, Wd2, bd2, x0p, True, Cp=16, Fp=8)
    return _layer(h, idxs, Wf, bf, None, False, Cp=8, out_wide=False)


# speedup vs baseline: 2.0771x; 1.0947x over previous
import functools

import jax
import jax.numpy as jnp
from jax import lax
from jax.experimental import pallas as pl
from jax.experimental.pallas import tpu as pltpu
from jax.experimental.pallas import tpu_sc as plsc

_NC = 2
_NS = 16
_NW = _NC * _NS

_LANES = 128
_IDX_CHUNK = 128
_HALF = 512
_SUPER = 1024


def _sc_gather(table_wide, idxq, Cp):
    M = idxq.shape[0]
    E = table_wide.shape[0]
    q = _LANES // Cp
    table = jnp.reshape(table_wide, (E * q, Cp))
    per_w = M // _NW
    supers = per_w // _SUPER
    mesh = plsc.VectorSubcoreMesh(
        core_axis_name="c", subcore_axis_name="s",
        num_cores=_NC, num_subcores=_NS)

    @functools.partial(
        pl.kernel,
        out_type=jax.ShapeDtypeStruct((M, _LANES), jnp.float32),
        mesh=mesh,
        scratch_types=[
            pltpu.VMEM((_HALF,), jnp.int32),
            pltpu.VMEM((_HALF,), jnp.int32),
            pltpu.VMEM((_HALF, Cp), jnp.float32),
            pltpu.VMEM((_HALF, Cp), jnp.float32),
            pltpu.SemaphoreType.DMA,
            pltpu.SemaphoreType.DMA,
        ],
        compiler_params=pltpu.CompilerParams(use_tc_tiling_on_sc=False),
    )
    def gather_kernel(table_hbm, idx_hbm, out_hbm, idx_v0, idx_v1,
                      rows_v0, rows_v1, sem0, sem1):
        wid = lax.axis_index("s") * _NC + lax.axis_index("c")
        base = wid * per_w
        idx_b = (idx_v0, idx_v1)
        rows_b = (rows_v0, rows_v1)
        sems = (sem0, sem1)

        def stage(half, buf):
            r0 = pl.multiple_of(base + half * _HALF, _HALF)
            pltpu.sync_copy(idx_hbm.at[pl.ds(r0, _HALF)], idx_b[buf])
            for j in range(_HALF // _IDX_CHUNK):
                pltpu.async_copy(
                    table_hbm.at[idx_b[buf].at[pl.ds(j * _IDX_CHUNK,
                                                     _IDX_CHUNK)]],
                    rows_b[buf].at[pl.ds(j * _IDX_CHUNK, _IDX_CHUNK)],
                    sems[buf])

        def drain(half, buf):
            pltpu.make_async_copy(
                table_hbm.at[pl.ds(0, _HALF)], rows_b[buf], sems[buf]).wait()
            r0 = pl.multiple_of(base + half * _HALF, _HALF)
            pltpu.sync_copy(
                rows_b[buf],
                out_hbm.at[pl.ds(r0, _HALF), pl.ds(0, Cp)])

        halves = per_w // _HALF

        def body(i, carry):
            h0 = 2 * i
            stage(h0, 0)

            @pl.when(i > 0)
            def _():
                drain(h0 - 1, 1)

            stage(h0 + 1, 1)
            drain(h0, 0)
            return carry

        lax.fori_loop(0, halves // 2, body, 0)
        drain(halves - 1, 1)

    return gather_kernel(table, idxq)


def _tc_conv(xp, g, Ws, bias, skip, act, out_wide):
    E = xp.shape[0]
    Cp, Fp = Ws[0].shape
    R = 4096
    grid = (E // R,)
    nb = E // R
    row_spec = pl.BlockSpec((R, _LANES), lambda i: (i, 0))
    g_specs = [pl.BlockSpec((R, _LANES), lambda i, k=k: (i + k * nb, 0))
               for k in range(4)]
    w_specs = [pl.BlockSpec((Cp, Fp), lambda i: (0, 0)) for _ in range(5)]
    b_spec = pl.BlockSpec((1, Fp), lambda i: (0, 0))
    out_w = _LANES if out_wide else Fp
    out_spec = pl.BlockSpec((R, out_w), lambda i: (i, 0))
    out_shape = jax.ShapeDtypeStruct((E, out_w), jnp.float32)

    def body(x_ref, a_ref, b_ref, c_ref, d_ref, w0, w1, w2, w3, w4,
             bias_ref, *rest):
        out_ref = rest[-1]
        xx = x_ref[...][:, :Cp]
        a = a_ref[...][:, :Cp]
        b = b_ref[...][:, :Cp]
        c = c_ref[...][:, :Cp]
        d = d_ref[...][:, :Cp]
        dot = functools.partial(
            jnp.dot, preferred_element_type=jnp.float32)
        h = dot(xx, w0[...])
        h += dot(jnp.abs(a - c), w1[...])
        h += dot(a + c, w2[...])
        h += dot(jnp.abs(b - d), w3[...])
        h += dot(b + d, w4[...])
        h += bias_ref[...]
        if act:
            h = jnp.where(h >= 0, h, 0.1 * h)
        if len(rest) == 2:
            h += rest[0][...][:, :Fp]
        if out_wide:
            h = jnp.pad(h, ((0, 0), (0, _LANES - Fp)))
        out_ref[...] = h

    in_specs = [row_spec] + g_specs + w_specs + [b_spec]
    args = [xp, g, g, g, g] + list(Ws) + [bias.reshape(1, Fp)]
    if skip is not None:
        in_specs.append(pl.BlockSpec((R, _LANES), lambda i: (i, 0)))
        args.append(skip)
    return pl.pallas_call(
        body,
        grid=grid,
        in_specs=in_specs,
        out_specs=out_spec,
        out_shape=out_shape,
    )(*args)


def _layer(hp, idxs, W, bias, skip, act, Cp, Fp=None, out_wide=True):
    C = W.shape[0] // 5
    F = W.shape[1]
    if Fp is None:
        Fp = F
    Ws = [jnp.pad(W[k * C:(k + 1) * C], ((0, Cp - C), (0, Fp - F)))
          for k in range(5)]
    biasp = jnp.pad(bias, (0, Fp - F))
    g = _sc_gather(hp, idxs[Cp], Cp)
    return _tc_conv(hp, g, Ws, biasp, skip, act, out_wide)


def kernel(fixed_input_features, gemm_edges, We0, be0, We1, be1, We2, be2,
           Wd0, bd0, Wd1, bd1, Wd2, bd2, Wf, bf):
    E = fixed_input_features.shape[0]
    idx = gemm_edges.T.reshape(4 * E)
    idxs = {cp: idx * (_LANES // cp) for cp in (8, 16, 32, 64)}

    x0p = jnp.pad(fixed_input_features, ((0, 0), (0, _LANES - 6)))
    s1 = _layer(x0p, idxs, We0, be0, None, True, Cp=8)
    s2 = _layer(s1, idxs, We1, be1, None, True, Cp=16)
    h = _layer(s2, idxs, We2, be2, None, True, Cp=32)
    h = _layer(h, idxs, Wd0, bd0, s2, True, Cp=64)
    h = _layer(h, idxs, Wd1, bd1, s1, True, Cp=32)
    h = _layer(h, idxs, Wd2, bd2, x0p, True, Cp=16, Fp=8)
    return _layer(h, idxs, Wf, bf, None, False, Cp=8, out_wide=False)


# speedup vs baseline: 2.1453x; 1.0329x over previous
import functools

import jax
import jax.numpy as jnp
from jax import lax
from jax.experimental import pallas as pl
from jax.experimental.pallas import tpu as pltpu
from jax.experimental.pallas import tpu_sc as plsc

_NC = 2
_NS = 16
_NW = _NC * _NS

_LANES = 128
_IDX_CHUNK = 128
_HALF = 512
_SUPER = 1024


def _sc_gather(table_wide, idxq, Cp):
    M = idxq.shape[0]
    E = table_wide.shape[0]
    q = _LANES // Cp
    table = jnp.reshape(table_wide, (E * q, Cp))
    per_w = M // _NW
    supers = per_w // _SUPER
    mesh = plsc.VectorSubcoreMesh(
        core_axis_name="c", subcore_axis_name="s",
        num_cores=_NC, num_subcores=_NS)

    @functools.partial(
        pl.kernel,
        out_type=jax.ShapeDtypeStruct((M, _LANES), jnp.float32),
        mesh=mesh,
        scratch_types=[
            pltpu.VMEM((_HALF,), jnp.int32),
            pltpu.VMEM((_HALF,), jnp.int32),
            pltpu.VMEM((_HALF, Cp), jnp.float32),
            pltpu.VMEM((_HALF, Cp), jnp.float32),
            pltpu.SemaphoreType.DMA,
            pltpu.SemaphoreType.DMA,
        ],
        compiler_params=pltpu.CompilerParams(use_tc_tiling_on_sc=False),
    )
    def gather_kernel(table_hbm, idx_hbm, out_hbm, idx_v0, idx_v1,
                      rows_v0, rows_v1, sem0, sem1):
        wid = lax.axis_index("s") * _NC + lax.axis_index("c")
        base = wid * per_w
        idx_b = (idx_v0, idx_v1)
        rows_b = (rows_v0, rows_v1)
        sems = (sem0, sem1)

        def stage(half, buf):
            r0 = pl.multiple_of(base + half * _HALF, _HALF)
            pltpu.sync_copy(idx_hbm.at[pl.ds(r0, _HALF)], idx_b[buf])
            for j in range(_HALF // _IDX_CHUNK):
                pltpu.async_copy(
                    table_hbm.at[idx_b[buf].at[pl.ds(j * _IDX_CHUNK,
                                                     _IDX_CHUNK)]],
                    rows_b[buf].at[pl.ds(j * _IDX_CHUNK, _IDX_CHUNK)],
                    sems[buf])

        def drain(half, buf):
            pltpu.make_async_copy(
                table_hbm.at[pl.ds(0, _HALF)], rows_b[buf], sems[buf]).wait()
            r0 = pl.multiple_of(base + half * _HALF, _HALF)
            pltpu.sync_copy(
                rows_b[buf],
                out_hbm.at[pl.ds(r0, _HALF), pl.ds(0, Cp)])

        halves = per_w // _HALF

        def body(i, carry):
            h0 = 2 * i
            stage(h0, 0)

            @pl.when(i > 0)
            def _():
                drain(h0 - 1, 1)

            stage(h0 + 1, 1)
            drain(h0, 0)
            return carry

        lax.fori_loop(0, halves // 2, body, 0)
        drain(halves - 1, 1)

    return gather_kernel(table, idxq)


def _tc_conv(xp, g, Ws, bias, skip, act, out_wide, stripe=0, nstripes=1,
             prev=None):
    E = xp.shape[0]
    Cp, Fp = Ws[0].shape
    R = 4096
    Es = E // nstripes
    grid = (Es // R,)
    nb = Es // R
    off = stripe * nb
    row_spec = pl.BlockSpec((R, _LANES), lambda i: (i + off, 0))
    g_specs = [pl.BlockSpec((R, _LANES), lambda i, k=k: (i + k * nb, 0))
               for k in range(4)]
    w_specs = [pl.BlockSpec((Cp, Fp), lambda i: (0, 0)) for _ in range(5)]
    b_spec = pl.BlockSpec((1, Fp), lambda i: (0, 0))
    out_w = _LANES if out_wide else Fp
    out_spec = pl.BlockSpec((R, out_w), lambda i: (i + off, 0))
    out_shape = jax.ShapeDtypeStruct((E, out_w), jnp.float32)

    has_skip = skip is not None

    def body(x_ref, a_ref, b_ref, c_ref, d_ref, w0, w1, w2, w3, w4,
             bias_ref, *rest):
        out_ref = rest[-1]
        xx = x_ref[...][:, :Cp]
        a = a_ref[...][:, :Cp]
        b = b_ref[...][:, :Cp]
        c = c_ref[...][:, :Cp]
        d = d_ref[...][:, :Cp]
        dot = functools.partial(
            jnp.dot, preferred_element_type=jnp.float32)
        h = dot(xx, w0[...])
        h += dot(jnp.abs(a - c), w1[...])
        h += dot(a + c, w2[...])
        h += dot(jnp.abs(b - d), w3[...])
        h += dot(b + d, w4[...])
        h += bias_ref[...]
        if act:
            h = jnp.where(h >= 0, h, 0.1 * h)
        if has_skip:
            h += rest[0][...][:, :Fp]
        if out_wide:
            h = jnp.pad(h, ((0, 0), (0, _LANES - Fp)))
        out_ref[...] = h

    in_specs = [row_spec] + g_specs + w_specs + [b_spec]
    args = [xp, g, g, g, g] + list(Ws) + [bias.reshape(1, Fp)]
    if skip is not None:
        in_specs.append(pl.BlockSpec((R, _LANES), lambda i: (i + off, 0)))
        args.append(skip)
    aliases = {}
    if prev is not None:
        aliases = {len(args): 0}
        in_specs.append(pl.BlockSpec(memory_space=pl.ANY))
        args.append(prev)
    return pl.pallas_call(
        body,
        grid=grid,
        in_specs=in_specs,
        out_specs=out_spec,
        out_shape=out_shape,
        input_output_aliases=aliases,
    )(*args)


_NSTRIPES = 2


def _layer(hp, idxs, W, bias, skip, act, Cp, Fp=None, out_wide=True):
    C = W.shape[0] // 5
    F = W.shape[1]
    if Fp is None:
        Fp = F
    Ws = [jnp.pad(W[k * C:(k + 1) * C], ((0, Cp - C), (0, Fp - F)))
          for k in range(5)]
    biasp = jnp.pad(bias, (0, Fp - F))
    gs = [_sc_gather(hp, idxs[Cp][s], Cp) for s in range(_NSTRIPES)]
    out = None
    for s in range(_NSTRIPES):
        out = _tc_conv(hp, gs[s], Ws, biasp, skip, act, out_wide,
                       stripe=s, nstripes=_NSTRIPES, prev=out)
    return out


def kernel(fixed_input_features, gemm_edges, We0, be0, We1, be1, We2, be2,
           Wd0, bd0, Wd1, bd1, Wd2, bd2, Wf, bf):
    E = fixed_input_features.shape[0]
    idx4 = gemm_edges.T
    Es = E // _NSTRIPES
    idxs = {cp: [(idx4[:, s * Es:(s + 1) * Es] * (_LANES // cp)).reshape(4 * Es)
                 for s in range(_NSTRIPES)]
            for cp in (8, 16, 32, 64)}

    x0p = jnp.pad(fixed_input_features, ((0, 0), (0, _LANES - 6)))
    s1 = _layer(x0p, idxs, We0, be0, None, True, Cp=8)
    s2 = _layer(s1, idxs, We1, be1, None, True, Cp=16)
    h = _layer(s2, idxs, We2, be2, None, True, Cp=32)
    h = _layer(h, idxs, Wd0, bd0, s2, True, Cp=64)
    h = _layer(h, idxs, Wd1, bd1, s1, True, Cp=32)
    h = _layer(h, idxs, Wd2, bd2, x0p, True, Cp=16, Fp=8)
    return _layer(h, idxs, Wf, bf, None, False, Cp=8, out_wide=False)


# speedup vs baseline: 2.2014x; 1.0261x over previous
import functools

import jax
import jax.numpy as jnp
from jax import lax
from jax.experimental import pallas as pl
from jax.experimental.pallas import tpu as pltpu
from jax.experimental.pallas import tpu_sc as plsc

_NC = 2
_NS = 16
_NW = _NC * _NS

_LANES = 128
_IDX_CHUNK = 128
_HALF = 512
_SUPER = 1024


def _sc_gather(table_wide, idxq, Cp):
    M = idxq.shape[0]
    E = table_wide.shape[0]
    q = _LANES // Cp
    table = jnp.reshape(table_wide, (E * q, Cp))
    per_w = M // _NW
    half = 512 if Cp == 64 else 1024
    mesh = plsc.VectorSubcoreMesh(
        core_axis_name="c", subcore_axis_name="s",
        num_cores=_NC, num_subcores=_NS)

    @functools.partial(
        pl.kernel,
        out_type=jax.ShapeDtypeStruct((M, _LANES), jnp.float32),
        mesh=mesh,
        scratch_types=[
            pltpu.VMEM((per_w,), jnp.int32),
            pltpu.VMEM((half, Cp), jnp.float32),
            pltpu.VMEM((half, Cp), jnp.float32),
            pltpu.SemaphoreType.DMA,
            pltpu.SemaphoreType.DMA,
        ],
        compiler_params=pltpu.CompilerParams(use_tc_tiling_on_sc=False),
    )
    def gather_kernel(table_hbm, idx_hbm, out_hbm, idx_v,
                      rows_v0, rows_v1, sem0, sem1):
        wid = lax.axis_index("s") * _NC + lax.axis_index("c")
        base = wid * per_w
        rows_b = (rows_v0, rows_v1)
        sems = (sem0, sem1)

        pltpu.sync_copy(idx_hbm.at[pl.ds(pl.multiple_of(base, per_w), per_w)],
                        idx_v)

        def stage(h, buf):
            for j in range(half // _IDX_CHUNK):
                pltpu.async_copy(
                    table_hbm.at[idx_v.at[pl.ds(
                        pl.multiple_of(h * half + j * _IDX_CHUNK, _IDX_CHUNK),
                        _IDX_CHUNK)]],
                    rows_b[buf].at[pl.ds(j * _IDX_CHUNK, _IDX_CHUNK)],
                    sems[buf])

        def drain(h, buf):
            pltpu.make_async_copy(
                table_hbm.at[pl.ds(0, half)], rows_b[buf], sems[buf]).wait()
            r0 = pl.multiple_of(base + h * half, half)
            pltpu.sync_copy(
                rows_b[buf],
                out_hbm.at[pl.ds(r0, half), pl.ds(0, Cp)])

        halves = per_w // half

        def body(i, carry):
            h0 = 2 * i
            stage(h0, 0)

            @pl.when(i > 0)
            def _():
                drain(h0 - 1, 1)

            stage(h0 + 1, 1)
            drain(h0, 0)
            return carry

        lax.fori_loop(0, halves // 2, body, 0)
        drain(halves - 1, 1)

    return gather_kernel(table, idxq)


def _tc_conv(xp, g, Ws, bias, skip, act, out_wide, stripe=0, nstripes=1,
             prev=None):
    E = xp.shape[0]
    Cp, Fp = Ws[0].shape
    R = 4096
    Es = E // nstripes
    grid = (Es // R,)
    nb = Es // R
    off = stripe * nb
    row_spec = pl.BlockSpec((R, _LANES), lambda i: (i + off, 0))
    g_specs = [pl.BlockSpec((R, _LANES), lambda i, k=k: (i + k * nb, 0))
               for k in range(4)]
    w_specs = [pl.BlockSpec((Cp, Fp), lambda i: (0, 0)) for _ in range(5)]
    b_spec = pl.BlockSpec((1, Fp), lambda i: (0, 0))
    out_w = _LANES if out_wide else Fp
    out_spec = pl.BlockSpec((R, out_w), lambda i: (i + off, 0))
    out_shape = jax.ShapeDtypeStruct((E, out_w), jnp.float32)

    has_skip = skip is not None

    def body(x_ref, a_ref, b_ref, c_ref, d_ref, w0, w1, w2, w3, w4,
             bias_ref, *rest):
        out_ref = rest[-1]
        xx = x_ref[...][:, :Cp]
        a = a_ref[...][:, :Cp]
        b = b_ref[...][:, :Cp]
        c = c_ref[...][:, :Cp]
        d = d_ref[...][:, :Cp]
        dot = functools.partial(
            jnp.dot, preferred_element_type=jnp.float32)
        h = dot(xx, w0[...])
        h += dot(jnp.abs(a - c), w1[...])
        h += dot(a + c, w2[...])
        h += dot(jnp.abs(b - d), w3[...])
        h += dot(b + d, w4[...])
        h += bias_ref[...]
        if act:
            h = jnp.where(h >= 0, h, 0.1 * h)
        if has_skip:
            h += rest[0][...][:, :Fp]
        if out_wide:
            h = jnp.pad(h, ((0, 0), (0, _LANES - Fp)))
        out_ref[...] = h

    in_specs = [row_spec] + g_specs + w_specs + [b_spec]
    args = [xp, g, g, g, g] + list(Ws) + [bias.reshape(1, Fp)]
    if skip is not None:
        in_specs.append(pl.BlockSpec((R, _LANES), lambda i: (i + off, 0)))
        args.append(skip)
    aliases = {}
    if prev is not None:
        aliases = {len(args): 0}
        in_specs.append(pl.BlockSpec(memory_space=pl.ANY))
        args.append(prev)
    return pl.pallas_call(
        body,
        grid=grid,
        in_specs=in_specs,
        out_specs=out_spec,
        out_shape=out_shape,
        input_output_aliases=aliases,
    )(*args)


_NSTRIPES = 2


def _layer(hp, idxs, W, bias, skip, act, Cp, Fp=None, out_wide=True):
    C = W.shape[0] // 5
    F = W.shape[1]
    if Fp is None:
        Fp = F
    Ws = [jnp.pad(W[k * C:(k + 1) * C], ((0, Cp - C), (0, Fp - F)))
          for k in range(5)]
    biasp = jnp.pad(bias, (0, Fp - F))
    gs = [_sc_gather(hp, idxs[Cp][s], Cp) for s in range(_NSTRIPES)]
    out = None
    for s in range(_NSTRIPES):
        out = _tc_conv(hp, gs[s], Ws, biasp, skip, act, out_wide,
                       stripe=s, nstripes=_NSTRIPES, prev=out)
    return out


def kernel(fixed_input_features, gemm_edges, We0, be0, We1, be1, We2, be2,
           Wd0, bd0, Wd1, bd1, Wd2, bd2, Wf, bf):
    E = fixed_input_features.shape[0]
    idx4 = gemm_edges.T
    Es = E // _NSTRIPES
    idxs = {cp: [(idx4[:, s * Es:(s + 1) * Es] * (_LANES // cp)).reshape(4 * Es)
                 for s in range(_NSTRIPES)]
            for cp in (8, 16, 32, 64)}

    x0p = jnp.pad(fixed_input_features, ((0, 0), (0, _LANES - 6)))
    s1 = _layer(x0p, idxs, We0, be0, None, True, Cp=8)
    s2 = _layer(s1, idxs, We1, be1, None, True, Cp=16)
    h = _layer(s2, idxs, We2, be2, None, True, Cp=32)
    h = _layer(h, idxs, Wd0, bd0, s2, True, Cp=64)
    h = _layer(h, idxs, Wd1, bd1, s1, True, Cp=32)
    h = _layer(h, idxs, Wd2, bd2, x0p, True, Cp=16, Fp=8)
    return _layer(h, idxs, Wf, bf, None, False, Cp=8, out_wide=False)


# speedup vs baseline: 2.2253x; 1.0109x over previous
import functools

import jax
import jax.numpy as jnp
from jax import lax
from jax.experimental import pallas as pl
from jax.experimental.pallas import tpu as pltpu
from jax.experimental.pallas import tpu_sc as plsc

_NC = 2
_NS = 16
_NW = _NC * _NS

_LANES = 128
_IDX_CHUNK = 128
_HALF = 512
_SUPER = 1024


def _sc_gather(table_wide, idxq, Cp):
    M = idxq.shape[0]
    E = table_wide.shape[0]
    q = _LANES // Cp
    table = jnp.reshape(table_wide, (E * q, Cp))
    per_w = M // _NW
    half = 512 if Cp == 64 else 1024
    mesh = plsc.VectorSubcoreMesh(
        core_axis_name="c", subcore_axis_name="s",
        num_cores=_NC, num_subcores=_NS)

    @functools.partial(
        pl.kernel,
        out_type=jax.ShapeDtypeStruct((M, _LANES), jnp.float32),
        mesh=mesh,
        scratch_types=[
            pltpu.VMEM((per_w,), jnp.int32),
            pltpu.VMEM((half, Cp), jnp.float32),
            pltpu.VMEM((half, Cp), jnp.float32),
            pltpu.SemaphoreType.DMA,
            pltpu.SemaphoreType.DMA,
        ],
        compiler_params=pltpu.CompilerParams(use_tc_tiling_on_sc=False),
    )
    def gather_kernel(table_hbm, idx_hbm, out_hbm, idx_v,
                      rows_v0, rows_v1, sem0, sem1):
        wid = lax.axis_index("s") * _NC + lax.axis_index("c")
        base = wid * per_w
        rows_b = (rows_v0, rows_v1)
        sems = (sem0, sem1)

        pltpu.sync_copy(idx_hbm.at[pl.ds(pl.multiple_of(base, per_w), per_w)],
                        idx_v)

        def stage(h, buf):
            for j in range(half // _IDX_CHUNK):
                pltpu.async_copy(
                    table_hbm.at[idx_v.at[pl.ds(
                        pl.multiple_of(h * half + j * _IDX_CHUNK, _IDX_CHUNK),
                        _IDX_CHUNK)]],
                    rows_b[buf].at[pl.ds(j * _IDX_CHUNK, _IDX_CHUNK)],
                    sems[buf])

        def drain(h, buf):
            pltpu.make_async_copy(
                table_hbm.at[pl.ds(0, half)], rows_b[buf], sems[buf]).wait()
            r0 = pl.multiple_of(base + h * half, half)
            pltpu.sync_copy(
                rows_b[buf],
                out_hbm.at[pl.ds(r0, half), pl.ds(0, Cp)])

        halves = per_w // half

        def body(i, carry):
            h0 = 2 * i
            stage(h0, 0)

            @pl.when(i > 0)
            def _():
                drain(h0 - 1, 1)

            stage(h0 + 1, 1)
            drain(h0, 0)
            return carry

        lax.fori_loop(0, halves // 2, body, 0)
        drain(halves - 1, 1)

    return gather_kernel(table, idxq)


def _tc_conv(xp, g, Ws, bias, skip, act, out_wide, stripe=0, nstripes=1,
             prev=None):
    E = xp.shape[0]
    Cp, Fp = Ws[0].shape
    R = 4096
    Es = E // nstripes
    grid = (Es // R,)
    nb = Es // R
    off = stripe * nb
    row_spec = pl.BlockSpec((R, _LANES), lambda i: (i + off, 0))
    g_specs = [pl.BlockSpec((R, _LANES), lambda i, k=k: (i + k * nb, 0))
               for k in range(4)]
    w_specs = [pl.BlockSpec((Cp, Fp), lambda i: (0, 0)) for _ in range(5)]
    b_spec = pl.BlockSpec((1, Fp), lambda i: (0, 0))
    out_w = _LANES if out_wide else Fp
    out_spec = pl.BlockSpec((R, out_w), lambda i: (i + off, 0))
    out_shape = jax.ShapeDtypeStruct((E, out_w), jnp.float32)

    has_skip = skip is not None

    def body(x_ref, a_ref, b_ref, c_ref, d_ref, w0, w1, w2, w3, w4,
             bias_ref, *rest):
        out_ref = rest[-1]
        xx = x_ref[...][:, :Cp]
        a = a_ref[...][:, :Cp]
        b = b_ref[...][:, :Cp]
        c = c_ref[...][:, :Cp]
        d = d_ref[...][:, :Cp]
        dot = functools.partial(
            jnp.dot, preferred_element_type=jnp.float32)
        h = dot(xx, w0[...])
        h += dot(jnp.abs(a - c), w1[...])
        h += dot(a + c, w2[...])
        h += dot(jnp.abs(b - d), w3[...])
        h += dot(b + d, w4[...])
        h += bias_ref[...]
        if act:
            h = jnp.where(h >= 0, h, 0.1 * h)
        if has_skip:
            h += rest[0][...][:, :Fp]
        if out_wide:
            h = jnp.pad(h, ((0, 0), (0, _LANES - Fp)))
        out_ref[...] = h

    in_specs = [row_spec] + g_specs + w_specs + [b_spec]
    args = [xp, g, g, g, g] + list(Ws) + [bias.reshape(1, Fp)]
    if skip is not None:
        in_specs.append(pl.BlockSpec((R, _LANES), lambda i: (i + off, 0)))
        args.append(skip)
    aliases = {}
    if prev is not None:
        aliases = {len(args): 0}
        in_specs.append(pl.BlockSpec(memory_space=pl.ANY))
        args.append(prev)
    return pl.pallas_call(
        body,
        grid=grid,
        in_specs=in_specs,
        out_specs=out_spec,
        out_shape=out_shape,
        input_output_aliases=aliases,
    )(*args)


_NSTRIPES = 1


def _layer(hp, idxs, W, bias, skip, act, Cp, Fp=None, out_wide=True):
    C = W.shape[0] // 5
    F = W.shape[1]
    if Fp is None:
        Fp = F
    Ws = [jnp.pad(W[k * C:(k + 1) * C], ((0, Cp - C), (0, Fp - F)))
          for k in range(5)]
    biasp = jnp.pad(bias, (0, Fp - F))
    gs = [_sc_gather(hp, idxs[Cp][s], Cp) for s in range(_NSTRIPES)]
    out = None
    for s in range(_NSTRIPES):
        out = _tc_conv(hp, gs[s], Ws, biasp, skip, act, out_wide,
                       stripe=s, nstripes=_NSTRIPES, prev=out)
    return out


def kernel(fixed_input_features, gemm_edges, We0, be0, We1, be1, We2, be2,
           Wd0, bd0, Wd1, bd1, Wd2, bd2, Wf, bf):
    E = fixed_input_features.shape[0]
    idx4 = gemm_edges.T
    Es = E // _NSTRIPES
    idxs = {cp: [(idx4[:, s * Es:(s + 1) * Es] * (_LANES // cp)).reshape(4 * Es)
                 for s in range(_NSTRIPES)]
            for cp in (8, 16, 32, 64)}

    x0p = jnp.pad(fixed_input_features, ((0, 0), (0, _LANES - 6)))
    s1 = _layer(x0p, idxs, We0, be0, None, True, Cp=8)
    s2 = _layer(s1, idxs, We1, be1, None, True, Cp=16)
    h = _layer(s2, idxs, We2, be2, None, True, Cp=32)
    h = _layer(h, idxs, Wd0, bd0, s2, True, Cp=64)
    h = _layer(h, idxs, Wd1, bd1, s1, True, Cp=32)
    h = _layer(h, idxs, Wd2, bd2, x0p, True, Cp=16, Fp=8)
    return _layer(h, idxs, Wf, bf, None, False, Cp=8, out_wide=False)


# speedup vs baseline: 2.7253x; 1.2247x over previous
import functools

import jax
import jax.numpy as jnp
from jax import lax
from jax.experimental import pallas as pl
from jax.experimental.pallas import tpu as pltpu
from jax.experimental.pallas import tpu_sc as plsc

_NC = 2
_NS = 16
_NW = _NC * _NS

_LANES = 128
_IDX_CHUNK = 128
_HALF = 512
_SUPER = 1024


def _sc_gather(table_wide, idxq, Cp):
    M = idxq.shape[0]
    E = table_wide.shape[0]
    q = _LANES // Cp
    table = jnp.reshape(table_wide, (E * q, Cp))
    per_w = M // _NW
    half = 512 if Cp == 64 else 1024
    mesh = plsc.VectorSubcoreMesh(
        core_axis_name="c", subcore_axis_name="s",
        num_cores=_NC, num_subcores=_NS)

    @functools.partial(
        pl.kernel,
        out_type=jax.ShapeDtypeStruct((M, _LANES), jnp.float32),
        mesh=mesh,
        scratch_types=[
            pltpu.VMEM((per_w,), jnp.int32),
            pltpu.VMEM((half, Cp), jnp.float32),
            pltpu.VMEM((half, Cp), jnp.float32),
            pltpu.SemaphoreType.DMA,
            pltpu.SemaphoreType.DMA,
        ],
        compiler_params=pltpu.CompilerParams(use_tc_tiling_on_sc=False),
    )
    def gather_kernel(table_hbm, idx_hbm, out_hbm, idx_v,
                      rows_v0, rows_v1, sem0, sem1):
        wid = lax.axis_index("s") * _NC + lax.axis_index("c")
        base = wid * per_w
        rows_b = (rows_v0, rows_v1)
        sems = (sem0, sem1)

        pltpu.sync_copy(idx_hbm.at[pl.ds(pl.multiple_of(base, per_w), per_w)],
                        idx_v)

        def stage(h, buf):
            for j in range(half // _IDX_CHUNK):
                pltpu.async_copy(
                    table_hbm.at[idx_v.at[pl.ds(
                        pl.multiple_of(h * half + j * _IDX_CHUNK, _IDX_CHUNK),
                        _IDX_CHUNK)]],
                    rows_b[buf].at[pl.ds(j * _IDX_CHUNK, _IDX_CHUNK)],
                    sems[buf])

        def drain(h, buf):
            pltpu.make_async_copy(
                table_hbm.at[pl.ds(0, half)], rows_b[buf], sems[buf]).wait()
            r0 = pl.multiple_of(base + h * half, half)
            pltpu.sync_copy(
                rows_b[buf],
                out_hbm.at[pl.ds(r0, half), pl.ds(0, Cp)])

        halves = per_w // half

        def body(i, carry):
            h0 = 2 * i
            stage(h0, 0)

            @pl.when(i > 0)
            def _():
                drain(h0 - 1, 1)

            stage(h0 + 1, 1)
            drain(h0, 0)
            return carry

        lax.fori_loop(0, halves // 2, body, 0)
        drain(halves - 1, 1)

    return gather_kernel(table, idxq)


def _sc_gather4(table_wide, idx4q, Cp):
    E = table_wide.shape[0]
    q = _LANES // Cp
    table = jnp.reshape(table_wide, (E * q, Cp))
    per_w = E // _NW
    half = 256
    sph = half // _IDX_CHUNK
    mesh = plsc.VectorSubcoreMesh(
        core_axis_name="c", subcore_axis_name="s",
        num_cores=_NC, num_subcores=_NS)

    @functools.partial(
        pl.kernel,
        out_type=jax.ShapeDtypeStruct((E, _LANES), jnp.float32),
        mesh=mesh,
        scratch_types=[
            pltpu.VMEM((4, per_w), jnp.int32),
            [pltpu.VMEM((half, Cp), jnp.float32) for _ in range(4)],
            [pltpu.VMEM((half, Cp), jnp.float32) for _ in range(4)],
            pltpu.SemaphoreType.DMA,
            pltpu.SemaphoreType.DMA,
        ],
        compiler_params=pltpu.CompilerParams(use_tc_tiling_on_sc=False),
    )
    def gather_kernel(table_hbm, idx_hbm, out_hbm, idx_v,
                      rows_v0, rows_v1, sem0, sem1):
        wid = lax.axis_index("s") * _NC + lax.axis_index("c")
        base = wid * per_w
        rows_b = (rows_v0, rows_v1)
        sems = (sem0, sem1)

        pltpu.sync_copy(
            idx_hbm.at[:, pl.ds(pl.multiple_of(base, per_w), per_w)], idx_v)

        def stage(h, buf):
            for k in range(4):
                for j in range(sph):
                    pltpu.async_copy(
                        table_hbm.at[idx_v.at[k, pl.ds(
                            pl.multiple_of(h * half + j * _IDX_CHUNK,
                                           _IDX_CHUNK),
                            _IDX_CHUNK)]],
                        rows_b[buf][k].at[pl.ds(j * _IDX_CHUNK, _IDX_CHUNK)],
                        sems[buf])

        def drain(h, buf):
            for k in range(4):
                pltpu.make_async_copy(
                    table_hbm.at[pl.ds(0, half)],
                    rows_b[buf][k], sems[buf]).wait()
            r0 = pl.multiple_of(base + h * half, half)
            for k in range(4):
                pltpu.sync_copy(
                    rows_b[buf][k],
                    out_hbm.at[pl.ds(r0, half), pl.ds(k * Cp, Cp)])

        halves = per_w // half

        def body(i, carry):
            h0 = 2 * i
            stage(h0, 0)

            @pl.when(i > 0)
            def _():
                drain(h0 - 1, 1)

            stage(h0 + 1, 1)
            drain(h0, 0)
            return carry

        lax.fori_loop(0, halves // 2, body, 0)
        drain(halves - 1, 1)

    return gather_kernel(table, idx4q)


def _tc_conv(xp, g, Ws, bias, skip, act, out_wide, stripe=0, nstripes=1,
             prev=None, packed=False):
    E = xp.shape[0]
    Cp, Fp = Ws[0].shape
    R = 4096
    Es = E // nstripes
    grid = (Es // R,)
    nb = Es // R
    off = stripe * nb
    row_spec = pl.BlockSpec((R, _LANES), lambda i: (i + off, 0))
    if packed:
        g_specs = [pl.BlockSpec((R, _LANES), lambda i: (i + off, 0))]
    else:
        g_specs = [pl.BlockSpec((R, _LANES), lambda i, k=k: (i + k * nb, 0))
                   for k in range(4)]
    w_specs = [pl.BlockSpec((Cp, Fp), lambda i: (0, 0)) for _ in range(5)]
    b_spec = pl.BlockSpec((1, Fp), lambda i: (0, 0))
    out_w = _LANES if out_wide else Fp
    out_spec = pl.BlockSpec((R, out_w), lambda i: (i + off, 0))
    out_shape = jax.ShapeDtypeStruct((E, out_w), jnp.float32)

    has_skip = skip is not None

    def body(x_ref, *refs):
        if packed:
            g_ref, = refs[:1]
            w0, w1, w2, w3, w4, bias_ref, *rest = refs[1:]
            gblk = g_ref[...]
            a = gblk[:, 0:Cp]
            c = gblk[:, Cp:2 * Cp]
            b = gblk[:, 2 * Cp:3 * Cp]
            d = gblk[:, 3 * Cp:4 * Cp]
        else:
            a_ref, b_ref, c_ref, d_ref = refs[:4]
            w0, w1, w2, w3, w4, bias_ref, *rest = refs[4:]
            a = a_ref[...][:, :Cp]
            b = b_ref[...][:, :Cp]
            c = c_ref[...][:, :Cp]
            d = d_ref[...][:, :Cp]
        out_ref = rest[-1]
        xx = x_ref[...][:, :Cp]
        dot = functools.partial(
            jnp.dot, preferred_element_type=jnp.float32)
        h = dot(xx, w0[...])
        h += dot(jnp.abs(a - c), w1[...])
        h += dot(a + c, w2[...])
        h += dot(jnp.abs(b - d), w3[...])
        h += dot(b + d, w4[...])
        h += bias_ref[...]
        if act:
            h = jnp.where(h >= 0, h, 0.1 * h)
        if has_skip:
            h += rest[0][...][:, :Fp]
        if out_wide:
            h = jnp.pad(h, ((0, 0), (0, _LANES - Fp)))
        out_ref[...] = h

    in_specs = [row_spec] + g_specs + w_specs + [b_spec]
    g_args = [g] if packed else [g, g, g, g]
    args = [xp] + g_args + list(Ws) + [bias.reshape(1, Fp)]
    if skip is not None:
        in_specs.append(pl.BlockSpec((R, _LANES), lambda i: (i + off, 0)))
        args.append(skip)
    aliases = {}
    if prev is not None:
        aliases = {len(args): 0}
        in_specs.append(pl.BlockSpec(memory_space=pl.ANY))
        args.append(prev)
    return pl.pallas_call(
        body,
        grid=grid,
        in_specs=in_specs,
        out_specs=out_spec,
        out_shape=out_shape,
        input_output_aliases=aliases,
    )(*args)


_NSTRIPES = 1


def _layer(hp, idxs, W, bias, skip, act, Cp, Fp=None, out_wide=True):
    C = W.shape[0] // 5
    F = W.shape[1]
    if Fp is None:
        Fp = F
    Ws = [jnp.pad(W[k * C:(k + 1) * C], ((0, Cp - C), (0, Fp - F)))
          for k in range(5)]
    biasp = jnp.pad(bias, (0, Fp - F))
    packed = Cp <= 32
    if packed:
        g = _sc_gather4(hp, idxs[Cp], Cp)
    else:
        g = _sc_gather(hp, idxs[Cp], Cp)
    return _tc_conv(hp, g, Ws, biasp, skip, act, out_wide, packed=packed)


def kernel(fixed_input_features, gemm_edges, We0, be0, We1, be1, We2, be2,
           Wd0, bd0, Wd1, bd1, Wd2, bd2, Wf, bf):
    E = fixed_input_features.shape[0]
    idx4 = gemm_edges.T
    idx4_acbd = idx4[jnp.array([0, 2, 1, 3])]
    idxs = {cp: idx4_acbd * (_LANES // cp) for cp in (8, 16, 32)}
    idxs[64] = idx4.reshape(4 * E) * (_LANES // 64)

    x0p = jnp.pad(fixed_input_features, ((0, 0), (0, _LANES - 6)))
    s1 = _layer(x0p, idxs, We0, be0, None, True, Cp=8)
    s2 = _layer(s1, idxs, We1, be1, None, True, Cp=16)
    h = _layer(s2, idxs, We2, be2, None, True, Cp=32)
    h = _layer(h, idxs, Wd0, bd0, s2, True, Cp=64)
    h = _layer(h, idxs, Wd1, bd1, s1, True, Cp=32)
    h = _layer(h, idxs, Wd2, bd2, x0p, True, Cp=16, Fp=8)
    return _layer(h, idxs, Wf, bf, None, False, Cp=8, out_wide=False)


# speedup vs baseline: 2.8195x; 1.0346x over previous
import functools

import jax
import jax.numpy as jnp
from jax import lax
from jax.experimental import pallas as pl
from jax.experimental.pallas import tpu as pltpu
from jax.experimental.pallas import tpu_sc as plsc

_NC = 2
_NS = 16
_NW = _NC * _NS

_LANES = 128
_IDX_CHUNK = 128
_HALF = 512
_SUPER = 1024


def _sc_gather(table_wide, idxq, Cp):
    M = idxq.shape[0]
    E = table_wide.shape[0]
    q = _LANES // Cp
    table = jnp.reshape(table_wide, (E * q, Cp))
    per_w = M // _NW
    half = 512 if Cp == 64 else 1024
    mesh = plsc.VectorSubcoreMesh(
        core_axis_name="c", subcore_axis_name="s",
        num_cores=_NC, num_subcores=_NS)

    @functools.partial(
        pl.kernel,
        out_type=jax.ShapeDtypeStruct((M, _LANES), jnp.float32),
        mesh=mesh,
        scratch_types=[
            pltpu.VMEM((per_w,), jnp.int32),
            pltpu.VMEM((half, Cp), jnp.float32),
            pltpu.VMEM((half, Cp), jnp.float32),
            pltpu.SemaphoreType.DMA,
            pltpu.SemaphoreType.DMA,
        ],
        compiler_params=pltpu.CompilerParams(use_tc_tiling_on_sc=False),
    )
    def gather_kernel(table_hbm, idx_hbm, out_hbm, idx_v,
                      rows_v0, rows_v1, sem0, sem1):
        wid = lax.axis_index("s") * _NC + lax.axis_index("c")
        base = wid * per_w
        rows_b = (rows_v0, rows_v1)
        sems = (sem0, sem1)

        pltpu.sync_copy(idx_hbm.at[pl.ds(pl.multiple_of(base, per_w), per_w)],
                        idx_v)

        def stage(h, buf):
            for j in range(half // _IDX_CHUNK):
                pltpu.async_copy(
                    table_hbm.at[idx_v.at[pl.ds(
                        pl.multiple_of(h * half + j * _IDX_CHUNK, _IDX_CHUNK),
                        _IDX_CHUNK)]],
                    rows_b[buf].at[pl.ds(j * _IDX_CHUNK, _IDX_CHUNK)],
                    sems[buf])

        def drain(h, buf):
            pltpu.make_async_copy(
                table_hbm.at[pl.ds(0, half)], rows_b[buf], sems[buf]).wait()
            r0 = pl.multiple_of(base + h * half, half)
            pltpu.sync_copy(
                rows_b[buf],
                out_hbm.at[pl.ds(r0, half), pl.ds(0, Cp)])

        halves = per_w // half

        def body(i, carry):
            h0 = 2 * i
            stage(h0, 0)

            @pl.when(i > 0)
            def _():
                drain(h0 - 1, 1)

            stage(h0 + 1, 1)
            drain(h0, 0)
            return carry

        lax.fori_loop(0, halves // 2, body, 0)
        drain(halves - 1, 1)

    return gather_kernel(table, idxq)


def _sc_gather4(table_wide, idx4q, Cp):
    E = table_wide.shape[0]
    q = _LANES // Cp
    spr = 4 if Cp <= 32 else 2
    table = jnp.reshape(table_wide, (E * q, Cp))
    OM = E * 4 // spr
    per_w = OM // _NW
    half = 256
    sph = half // _IDX_CHUNK
    mesh = plsc.VectorSubcoreMesh(
        core_axis_name="c", subcore_axis_name="s",
        num_cores=_NC, num_subcores=_NS)

    @functools.partial(
        pl.kernel,
        out_type=jax.ShapeDtypeStruct((OM, _LANES), jnp.float32),
        mesh=mesh,
        scratch_types=[
            pltpu.VMEM((spr, per_w), jnp.int32),
            [pltpu.VMEM((half, Cp), jnp.float32) for _ in range(spr)],
            [pltpu.VMEM((half, Cp), jnp.float32) for _ in range(spr)],
            pltpu.SemaphoreType.DMA,
            pltpu.SemaphoreType.DMA,
        ],
        compiler_params=pltpu.CompilerParams(use_tc_tiling_on_sc=False),
    )
    def gather_kernel(table_hbm, idx_hbm, out_hbm, idx_v,
                      rows_v0, rows_v1, sem0, sem1):
        wid = lax.axis_index("s") * _NC + lax.axis_index("c")
        base = wid * per_w
        rows_b = (rows_v0, rows_v1)
        sems = (sem0, sem1)

        if spr == 4:
            pltpu.sync_copy(
                idx_hbm.at[:, pl.ds(pl.multiple_of(base, per_w), per_w)],
                idx_v)
        else:
            pair = wid // (_NW // 2)
            col = (wid % (_NW // 2)) * per_w
            pltpu.sync_copy(
                idx_hbm.at[pl.ds(pl.multiple_of(2 * pair, 2), 2),
                           pl.ds(pl.multiple_of(col, per_w), per_w)],
                idx_v)

        def stage(h, buf):
            for k in range(spr):
                for j in range(sph):
                    pltpu.async_copy(
                        table_hbm.at[idx_v.at[k, pl.ds(
                            pl.multiple_of(h * half + j * _IDX_CHUNK,
                                           _IDX_CHUNK),
                            _IDX_CHUNK)]],
                        rows_b[buf][k].at[pl.ds(j * _IDX_CHUNK, _IDX_CHUNK)],
                        sems[buf])

        def drain(h, buf):
            for k in range(spr):
                pltpu.make_async_copy(
                    table_hbm.at[pl.ds(0, half)],
                    rows_b[buf][k], sems[buf]).wait()
            r0 = pl.multiple_of(base + h * half, half)
            for k in range(spr):
                pltpu.sync_copy(
                    rows_b[buf][k],
                    out_hbm.at[pl.ds(r0, half), pl.ds(k * Cp, Cp)])

        halves = per_w // half

        def body(i, carry):
            h0 = 2 * i
            stage(h0, 0)

            @pl.when(i > 0)
            def _():
                drain(h0 - 1, 1)

            stage(h0 + 1, 1)
            drain(h0, 0)
            return carry

        lax.fori_loop(0, halves // 2, body, 0)
        drain(halves - 1, 1)

    return gather_kernel(table, idx4q)


def _tc_conv(xp, g, Ws, bias, skip, act, out_wide, stripe=0, nstripes=1,
             prev=None, packed=False):
    E = xp.shape[0]
    Cp, Fp = Ws[0].shape
    R = 4096
    Es = E // nstripes
    grid = (Es // R,)
    nb = Es // R
    off = stripe * nb
    row_spec = pl.BlockSpec((R, _LANES), lambda i: (i + off, 0))
    if packed == 4:
        g_specs = [pl.BlockSpec((R, _LANES), lambda i: (i + off, 0))]
    else:
        g_specs = [pl.BlockSpec((R, _LANES), lambda i: (i + off, 0)),
                   pl.BlockSpec((R, _LANES), lambda i: (i + off + E // R, 0))]
    w_specs = [pl.BlockSpec((Cp, Fp), lambda i: (0, 0)) for _ in range(5)]
    b_spec = pl.BlockSpec((1, Fp), lambda i: (0, 0))
    out_w = _LANES if out_wide else Fp
    out_spec = pl.BlockSpec((R, out_w), lambda i: (i + off, 0))
    out_shape = jax.ShapeDtypeStruct((E, out_w), jnp.float32)

    has_skip = skip is not None

    def body(x_ref, *refs):
        if packed == 4:
            g_ref, = refs[:1]
            w0, w1, w2, w3, w4, bias_ref, *rest = refs[1:]
            gblk = g_ref[...]
            a = gblk[:, 0:Cp]
            c = gblk[:, Cp:2 * Cp]
            b = gblk[:, 2 * Cp:3 * Cp]
            d = gblk[:, 3 * Cp:4 * Cp]
        else:
            gac_ref, gbd_ref = refs[:2]
            w0, w1, w2, w3, w4, bias_ref, *rest = refs[2:]
            gac = gac_ref[...]
            gbd = gbd_ref[...]
            a = gac[:, 0:Cp]
            c = gac[:, Cp:2 * Cp]
            b = gbd[:, 0:Cp]
            d = gbd[:, Cp:2 * Cp]
        out_ref = rest[-1]
        xx = x_ref[...][:, :Cp]
        dot = functools.partial(
            jnp.dot, preferred_element_type=jnp.float32)
        h = dot(xx, w0[...])
        h += dot(jnp.abs(a - c), w1[...])
        h += dot(a + c, w2[...])
        h += dot(jnp.abs(b - d), w3[...])
        h += dot(b + d, w4[...])
        h += bias_ref[...]
        if act:
            h = jnp.where(h >= 0, h, 0.1 * h)
        if has_skip:
            h += rest[0][...][:, :Fp]
        if out_wide:
            h = jnp.pad(h, ((0, 0), (0, _LANES - Fp)))
        out_ref[...] = h

    in_specs = [row_spec] + g_specs + w_specs + [b_spec]
    g_args = [g] if packed == 4 else [g, g]
    args = [xp] + g_args + list(Ws) + [bias.reshape(1, Fp)]
    if skip is not None:
        in_specs.append(pl.BlockSpec((R, _LANES), lambda i: (i + off, 0)))
        args.append(skip)
    aliases = {}
    if prev is not None:
        aliases = {len(args): 0}
        in_specs.append(pl.BlockSpec(memory_space=pl.ANY))
        args.append(prev)
    return pl.pallas_call(
        body,
        grid=grid,
        in_specs=in_specs,
        out_specs=out_spec,
        out_shape=out_shape,
        input_output_aliases=aliases,
    )(*args)


_NSTRIPES = 1


def _layer(hp, idxs, W, bias, skip, act, Cp, Fp=None, out_wide=True):
    C = W.shape[0] // 5
    F = W.shape[1]
    if Fp is None:
        Fp = F
    Ws = [jnp.pad(W[k * C:(k + 1) * C], ((0, Cp - C), (0, Fp - F)))
          for k in range(5)]
    biasp = jnp.pad(bias, (0, Fp - F))
    spr = 4 if Cp <= 32 else 2
    g = _sc_gather4(hp, idxs[Cp], Cp)
    return _tc_conv(hp, g, Ws, biasp, skip, act, out_wide, packed=spr)


def kernel(fixed_input_features, gemm_edges, We0, be0, We1, be1, We2, be2,
           Wd0, bd0, Wd1, bd1, Wd2, bd2, Wf, bf):
    E = fixed_input_features.shape[0]
    idx4 = gemm_edges.T
    idx4_acbd = idx4[jnp.array([0, 2, 1, 3])]
    idxs = {cp: idx4_acbd * (_LANES // cp) for cp in (8, 16, 32, 64)}

    x0p = jnp.pad(fixed_input_features, ((0, 0), (0, _LANES - 6)))
    s1 = _layer(x0p, idxs, We0, be0, None, True, Cp=8)
    s2 = _layer(s1, idxs, We1, be1, None, True, Cp=16)
    h = _layer(s2, idxs, We2, be2, None, True, Cp=32)
    h = _layer(h, idxs, Wd0, bd0, s2, True, Cp=64)
    h = _layer(h, idxs, Wd1, bd1, s1, True, Cp=32)
    h = _layer(h, idxs, Wd2, bd2, x0p, True, Cp=16, Fp=8)
    return _layer(h, idxs, Wf, bf, None, False, Cp=8, out_wide=False)
